# baseline (speedup 1.0000x reference)
import functools

import jax
import jax.numpy as jnp
from jax import lax
from jax.experimental import pallas as pl

_D = 1024
_R = 400


def _stage_body(x_ref, x2_ref, cb_ref, c2_ref,
                res_ref, qst_ref, codes_ref, loss_ref, *, ksz):
    x = x_ref[...]
    cb = cb_ref[...]
    dn = (((1,), (1,)), ((), ()))
    u = 2.0 * (lax.dot_general(x[:, :512], cb[:, :512], dn,
                               preferred_element_type=jnp.float32)
               + lax.dot_general(x[:, 512:], cb[:, 512:], dn,
                                 preferred_element_type=jnp.float32))
    d2 = (x2_ref[...] - u) + c2_ref[0:1, :]
    dd = jnp.sqrt(jnp.maximum(d2, 0.0))
    rowmin = jnp.min(dd, axis=1, keepdims=True)
    iota = lax.broadcasted_iota(jnp.int32, dd.shape, 1)
    idx = jnp.min(jnp.where(dd == rowmin, iota, ksz), axis=1,
                  keepdims=True)
    oh = (iota == idx).astype(jnp.float32)
    qz = lax.dot_general(oh, cb, (((1,), (0,)), ((), ())),
                         precision=lax.Precision.HIGHEST,
                         preferred_element_type=jnp.float32)
    t = qz - x
    q_st = x + t
    res_ref[...] = x - q_st
    qst_ref[...] = q_st
    codes_ref[...] = idx
    loss = jnp.sum(t * t, axis=(0, 1), keepdims=True)

    @pl.when(pl.program_id(0) == 0)
    def _first():
        loss_ref[...] = loss

    @pl.when(pl.program_id(0) != 0)
    def _rest():
        loss_ref[...] = loss_ref[...] + loss


def _stage(x, cb):
    n, Dv = x.shape
    ksz = cb.shape[0]
    x2 = jnp.sum(x * x, axis=1, keepdims=True)
    c2 = jnp.sum(cb * cb, axis=1)[None, :]
    row_spec = pl.BlockSpec((_R, _D), lambda i: (i, 0))
    return pl.pallas_call(
        functools.partial(_stage_body, ksz=ksz),
        grid=(n // _R,),
        in_specs=[row_spec,
                  pl.BlockSpec((_R, 1), lambda i: (i, 0)),
                  pl.BlockSpec((ksz, _D), lambda i: (0, 0)),
                  pl.BlockSpec((1, ksz), lambda i: (0, 0))],
        out_specs=[row_spec, row_spec,
                   pl.BlockSpec((_R, 1), lambda i: (i, 0)),
                   pl.BlockSpec((1, 1), lambda i: (0, 0))],
        out_shape=[
            jax.ShapeDtypeStruct((n, Dv), jnp.float32),
            jax.ShapeDtypeStruct((n, Dv), jnp.float32),
            jax.ShapeDtypeStruct((n, 1), jnp.int32),
            jax.ShapeDtypeStruct((1, 1), jnp.float32),
        ],
    )(x, x2, cb, c2)


def kernel(z, cb0, cb1, cb2, cb3, cb4, cb5, cb6, cb7):
    Bv, Dv, Tv = z.shape
    n = Bv * Tv
    zt = jnp.transpose(z, (0, 2, 1)).reshape(n, Dv)

    res = zt
    quant = jnp.zeros_like(zt)
    codes_cols = []
    loss = jnp.zeros((), jnp.float32)
    for cb in (cb0, cb1, cb2, cb3, cb4, cb5, cb6, cb7):
        res, q_st, idx, l = _stage(res, cb)
        quant = quant + q_st
        codes_cols.append(idx)
        loss = loss + l[0, 0]

    out = quant.reshape(Bv, Tv, Dv).transpose(0, 2, 1)
    codes = jnp.concatenate(codes_cols, axis=1)
    codes_arr = codes.reshape(Bv, Tv, 8).transpose(0, 2, 1)
    total_loss = loss * (1.25 / (8 * n * Dv))
    return out, codes_arr, total_loss
